# baseline (speedup 1.0000x reference)
import functools

import jax
import jax.numpy as jnp
from jax import lax
from jax.experimental import pallas as pl
from jax.experimental.pallas import tpu as pltpu
from jax.experimental.pallas import tpu_sc as plsc

B = 16384
D = 32

_info = plsc.get_sparse_core_info()
NC = _info.num_cores
NS = _info.num_subcores
L = _info.num_lanes
NW = NC * NS
BPW = B // NW
NG = BPW // L

_mesh = plsc.VectorSubcoreMesh(core_axis_name="c", subcore_axis_name="s")


@functools.partial(
    pl.kernel,
    mesh=_mesh,
    compiler_params=pltpu.CompilerParams(
        needs_layout_passes=False, use_tc_tiling_on_sc=False),
    out_type=jax.ShapeDtypeStruct((B,), jnp.float32),
    scratch_types=[
        pltpu.VMEM((BPW,), jnp.int32),
        pltpu.VMEM((BPW,), jnp.int32),
        pltpu.VMEM((BPW, D), jnp.float32),
        pltpu.VMEM((BPW, D), jnp.float32),
        pltpu.VMEM((BPW,), jnp.float32),
        pltpu.VMEM((BPW,), jnp.float32),
        pltpu.VMEM((BPW,), jnp.float32),
        pltpu.VMEM((BPW,), jnp.float32),
        pltpu.SemaphoreType.DMA,
        pltpu.SemaphoreType.DMA,
        pltpu.SemaphoreType.DMA,
        pltpu.SemaphoreType.DMA,
    ],
)
def _mf_kernel(uid_hbm, iid_hbm, avg_hbm, p_hbm, q_hbm, ub_hbm, ib_hbm,
               out_hbm, uidx, iidx, urows, qrows, ubv, ibv, avgv, outv,
               sem_p, sem_q, sem_ub, sem_ib):
    wid = lax.axis_index("s") * NC + lax.axis_index("c")
    base = wid * BPW

    pltpu.sync_copy(uid_hbm.at[pl.ds(base, BPW)], uidx)
    pltpu.sync_copy(iid_hbm.at[pl.ds(base, BPW)], iidx)
    cp = pltpu.async_copy(p_hbm.at[uidx], urows, sem_p)
    cq = pltpu.async_copy(q_hbm.at[iidx], qrows, sem_q)
    cu = pltpu.async_copy(ub_hbm.at[uidx], ubv, sem_ub)
    ci = pltpu.async_copy(ib_hbm.at[iidx], ibv, sem_ib)
    pltpu.sync_copy(avg_hbm.at[pl.ds(base, BPW)], avgv)
    cp.wait()
    cq.wait()
    cu.wait()
    ci.wait()

    lane = lax.iota(jnp.int32, L)

    def body(g, _):
        o = g * L
        acc = avgv[pl.ds(o, L)] + ubv[pl.ds(o, L)] + ibv[pl.ds(o, L)]
        rows = o + lane
        for d in range(D):
            col = jnp.full((L,), d, jnp.int32)
            acc += (plsc.load_gather(urows, [rows, col])
                    * plsc.load_gather(qrows, [rows, col]))
        outv[pl.ds(o, L)] = acc
        return 0

    lax.fori_loop(0, NG, body, 0)
    pltpu.sync_copy(outv, out_hbm.at[pl.ds(base, BPW)])


def kernel(user_id, item_id, avg_score, p, q, user_bias, item_bias):
    out = _mf_kernel(user_id, item_id, avg_score.reshape(B), p, q,
                     user_bias.reshape(-1), item_bias.reshape(-1))
    return out.reshape(B, 1)
